# baseline (speedup 1.0000x reference)
import functools

import jax
import jax.numpy as jnp
from jax import lax
from jax.experimental import pallas as pl
from jax.experimental.pallas import tpu as pltpu
from jax.experimental.pallas import tpu_sc as plsc

VOCAB = 1000000
EMB = 64
HID = 128
OUT = 2
L, B = 200, 4096

NC, NS = 2, 16
NW = NC * NS
BPW = B // NW
LANES = 16
INV_L = 1.0 / L

HALF_V = VOCAB // 2

RING = 4


def _mean_pool_body(x_hbm, tab_hbm, m_hbm, idx_v, rows_v, acc_v, *sems):
    wid = lax.axis_index("s") * NC + lax.axis_index("c")
    base = wid * BPW

    pltpu.sync_copy(x_hbm.at[:, pl.ds(base, BPW)], idx_v)

    def issue(l, s):
        pltpu.async_copy(tab_hbm.at[idx_v.at[l]], rows_v.at[s], sems[s])

    def drain(s):
        pltpu.make_async_copy(tab_hbm.at[pl.ds(0, BPW)],
                              rows_v.at[s], sems[s]).wait()

    def zero(r, carry):
        for c in range(EMB // LANES):
            acc_v[r, pl.ds(c * LANES, LANES)] = jnp.zeros((LANES,),
                                                          jnp.float32)
        return carry

    lax.fori_loop(0, BPW, zero, 0)

    for s in range(RING):
        issue(s, s)

    def outer(i, carry):
        for s in range(RING):
            l = i * RING + s
            drain(s)

            def red(r, carry2):
                for c in range(EMB // LANES):
                    plsc.addupdate(acc_v.at[r, pl.ds(c * LANES, LANES)],
                                   rows_v[s, r, pl.ds(c * LANES, LANES)])
                return carry2

            lax.fori_loop(0, BPW, red, 0)

            nl = l + RING

            @pl.when(nl < L)
            def _():
                issue(nl, s)
        return carry

    lax.fori_loop(0, L // RING, outer, 0)

    def scale(r, carry):
        for c in range(EMB // LANES):
            sl = pl.ds(c * LANES, LANES)
            acc_v[r, sl] = acc_v[r, sl] * INV_L
        return carry

    lax.fori_loop(0, BPW, scale, 0)
    pltpu.sync_copy(acc_v, m_hbm.at[pl.ds(base, BPW)])


@functools.partial(
    pl.kernel,
    out_type=jax.ShapeDtypeStruct((B, EMB), jnp.float32),
    mesh=plsc.VectorSubcoreMesh(core_axis_name="c", subcore_axis_name="s"),
    scratch_types=[
        pltpu.VMEM((L, BPW), jnp.int32),
        pltpu.VMEM((RING, BPW, EMB), jnp.float32),
        pltpu.VMEM((BPW, EMB), jnp.float32),
    ] + [pltpu.SemaphoreType.DMA] * RING,
    compiler_params=pltpu.CompilerParams(use_tc_tiling_on_sc=False),
)
def _mean_pool(x_hbm, tab_hbm, m_hbm, idx_v, rows_v, acc_v, *sems):
    _mean_pool_body(x_hbm, tab_hbm, m_hbm, idx_v, rows_v, acc_v, *sems)


def _mlp_body(m_ref, w1_ref, b1_ref, w2_ref, b2_ref, o_ref):
    h = jnp.dot(m_ref[...], w1_ref[...], preferred_element_type=jnp.float32)
    h = jnp.maximum(h + b1_ref[...], 0.0)
    o_ref[...] = jnp.dot(h, w2_ref[...],
                         preferred_element_type=jnp.float32) + b2_ref[...]


_mlp = pl.pallas_call(
    _mlp_body,
    out_shape=jax.ShapeDtypeStruct((B, OUT), jnp.float32),
)


def kernel(x, emb, W1, b1, W2, b2):
    packed = jax.lax.optimization_barrier(emb.reshape(HALF_V, 2 * EMB))
    tab = packed.reshape(VOCAB, EMB)
    m = _mean_pool(x, tab)
    return _mlp(m, W1, b1.reshape(1, HID), W2, b2.reshape(1, OUT))


# speedup vs baseline: 1.9981x; 1.9981x over previous
import functools

import jax
import jax.numpy as jnp
from jax import lax
from jax.experimental import pallas as pl
from jax.experimental.pallas import tpu as pltpu
from jax.experimental.pallas import tpu_sc as plsc

VOCAB = 1000000
EMB = 64
HID = 128
OUT = 2
L, B = 200, 4096

NC, NS = 2, 16
NW = NC * NS
BPW = B // NW
LANES = 16
INV_L = 1.0 / L

HALF_V = VOCAB // 2
PU = 64
HU = PU // 2
CBLK = 16384
TGRID = (VOCAB + CBLK - 1) // CBLK


def _tpack_body(a_ref, o_ref):
    t = a_ref[...].T
    u = t.reshape(CBLK // PU, 2, HU, EMB)
    o = jnp.concatenate([u[:, 0], u[:, 1]], axis=2)
    o_ref[...] = o.reshape(CBLK // 2, 2 * EMB)


_tpack = pl.pallas_call(
    _tpack_body,
    grid=(TGRID,),
    in_specs=[pl.BlockSpec((EMB, CBLK), lambda i: (0, i))],
    out_specs=pl.BlockSpec((CBLK // 2, 2 * EMB), lambda i: (i, 0)),
    out_shape=jax.ShapeDtypeStruct((HALF_V, 2 * EMB), jnp.float32),
)


RING = 4


def _mean_pool_body(x_hbm, tab_hbm, m_hbm, idx_v, rows_v, acc_v, *sems):
    wid = lax.axis_index("s") * NC + lax.axis_index("c")
    base = wid * BPW

    pltpu.sync_copy(x_hbm.at[:, pl.ds(base, BPW)], idx_v)

    def remap(l, carry):
        for c in range(BPW // LANES):
            sl = pl.ds(c * LANES, LANES)
            v = idx_v[l, sl]
            q = v & (PU - 1)
            idx_v[l, sl] = v + q + jnp.where(q >= HU, 1 - PU, 0)
        return carry

    lax.fori_loop(0, L, remap, 0)

    def issue(l, s):
        pltpu.async_copy(tab_hbm.at[idx_v.at[l]], rows_v.at[s], sems[s])

    def drain(s):
        pltpu.make_async_copy(tab_hbm.at[pl.ds(0, BPW)],
                              rows_v.at[s], sems[s]).wait()

    def zero(r, carry):
        for c in range(EMB // LANES):
            acc_v[r, pl.ds(c * LANES, LANES)] = jnp.zeros((LANES,),
                                                          jnp.float32)
        return carry

    lax.fori_loop(0, BPW, zero, 0)

    for s in range(RING):
        issue(s, s)

    def outer(i, carry):
        for s in range(RING):
            l = i * RING + s
            drain(s)

            def red(r, carry2):
                for c in range(EMB // LANES):
                    plsc.addupdate(acc_v.at[r, pl.ds(c * LANES, LANES)],
                                   rows_v[s, r, pl.ds(c * LANES, LANES)])
                return carry2

            lax.fori_loop(0, BPW, red, 0)

            nl = l + RING

            @pl.when(nl < L)
            def _():
                issue(nl, s)
        return carry

    lax.fori_loop(0, L // RING, outer, 0)

    def scale(r, carry):
        for c in range(EMB // LANES):
            sl = pl.ds(c * LANES, LANES)
            acc_v[r, sl] = acc_v[r, sl] * INV_L
        return carry

    lax.fori_loop(0, BPW, scale, 0)
    pltpu.sync_copy(acc_v, m_hbm.at[pl.ds(base, BPW)])


@functools.partial(
    pl.kernel,
    out_type=jax.ShapeDtypeStruct((B, EMB), jnp.float32),
    mesh=plsc.VectorSubcoreMesh(core_axis_name="c", subcore_axis_name="s"),
    scratch_types=[
        pltpu.VMEM((L, BPW), jnp.int32),
        pltpu.VMEM((RING, BPW, EMB), jnp.float32),
        pltpu.VMEM((BPW, EMB), jnp.float32),
    ] + [pltpu.SemaphoreType.DMA] * RING,
    compiler_params=pltpu.CompilerParams(use_tc_tiling_on_sc=False),
)
def _mean_pool(x_hbm, tab_hbm, m_hbm, idx_v, rows_v, acc_v, *sems):
    _mean_pool_body(x_hbm, tab_hbm, m_hbm, idx_v, rows_v, acc_v, *sems)


def _mlp_body(m_ref, w1_ref, b1_ref, w2_ref, b2_ref, o_ref):
    h = jnp.dot(m_ref[...], w1_ref[...], preferred_element_type=jnp.float32)
    h = jnp.maximum(h + b1_ref[...], 0.0)
    o_ref[...] = jnp.dot(h, w2_ref[...],
                         preferred_element_type=jnp.float32) + b2_ref[...]


_mlp = pl.pallas_call(
    _mlp_body,
    out_shape=jax.ShapeDtypeStruct((B, OUT), jnp.float32),
)


def kernel(x, emb, W1, b1, W2, b2):
    packed = _tpack(emb.T)
    tab = packed.reshape(VOCAB, EMB)
    m = _mean_pool(x, tab)
    return _mlp(m, W1, b1.reshape(1, HID), W2, b2.reshape(1, OUT))


# speedup vs baseline: 2.3450x; 1.1736x over previous
import functools

import jax
import jax.numpy as jnp
from jax import lax
from jax.experimental import pallas as pl
from jax.experimental.pallas import tpu as pltpu
from jax.experimental.pallas import tpu_sc as plsc

VOCAB = 1000000
EMB = 64
HID = 128
OUT = 2
L, B = 200, 4096

NC, NS = 2, 16
NW = NC * NS
BPW = B // NW
LANES = 16
INV_L = 1.0 / L

CBLK = 16384
PU = CBLK
HU = PU // 2
TGRID = (VOCAB + CBLK - 1) // CBLK
PACK_ROWS = TGRID * (CBLK // 2)
PACK_V = PACK_ROWS * 2


def _tpack_body(a_ref, o_ref):
    a = a_ref[...]
    c = jnp.concatenate([a[:, : CBLK // 2], a[:, CBLK // 2 :]], axis=0)
    o_ref[...] = c.T


_tpack = pl.pallas_call(
    _tpack_body,
    grid=(TGRID,),
    in_specs=[pl.BlockSpec((EMB, CBLK), lambda i: (0, i))],
    out_specs=pl.BlockSpec((CBLK // 2, 2 * EMB), lambda i: (i, 0)),
    out_shape=jax.ShapeDtypeStruct((PACK_ROWS, 2 * EMB), jnp.float32),
)


RING = 4


def _mean_pool_body(x_hbm, tab_hbm, m_hbm, idx_v, rows_v, acc_v, *sems):
    wid = lax.axis_index("s") * NC + lax.axis_index("c")
    base = wid * BPW

    pltpu.sync_copy(x_hbm.at[:, pl.ds(base, BPW)], idx_v)

    def remap(l, carry):
        for c in range(BPW // LANES):
            sl = pl.ds(c * LANES, LANES)
            v = idx_v[l, sl]
            q = v & (PU - 1)
            idx_v[l, sl] = v + q + jnp.where(q >= HU, 1 - PU, 0)
        return carry

    lax.fori_loop(0, L, remap, 0)

    def issue(l, s):
        pltpu.async_copy(tab_hbm.at[idx_v.at[l]], rows_v.at[s], sems[s])

    def drain(s):
        pltpu.make_async_copy(tab_hbm.at[pl.ds(0, BPW)],
                              rows_v.at[s], sems[s]).wait()

    def zero(r, carry):
        for c in range(EMB // LANES):
            acc_v[r, pl.ds(c * LANES, LANES)] = jnp.zeros((LANES,),
                                                          jnp.float32)
        return carry

    lax.fori_loop(0, BPW, zero, 0)

    for s in range(RING):
        issue(s, s)

    def outer(i, carry):
        for s in range(RING):
            l = i * RING + s
            drain(s)

            def red(r, carry2):
                for c in range(EMB // LANES):
                    plsc.addupdate(acc_v.at[r, pl.ds(c * LANES, LANES)],
                                   rows_v[s, r, pl.ds(c * LANES, LANES)])
                return carry2

            lax.fori_loop(0, BPW, red, 0)

            nl = l + RING

            @pl.when(nl < L)
            def _():
                issue(nl, s)
        return carry

    lax.fori_loop(0, L // RING, outer, 0)

    def scale(r, carry):
        for c in range(EMB // LANES):
            sl = pl.ds(c * LANES, LANES)
            acc_v[r, sl] = acc_v[r, sl] * INV_L
        return carry

    lax.fori_loop(0, BPW, scale, 0)
    pltpu.sync_copy(acc_v, m_hbm.at[pl.ds(base, BPW)])


@functools.partial(
    pl.kernel,
    out_type=jax.ShapeDtypeStruct((B, EMB), jnp.float32),
    mesh=plsc.VectorSubcoreMesh(core_axis_name="c", subcore_axis_name="s"),
    scratch_types=[
        pltpu.VMEM((L, BPW), jnp.int32),
        pltpu.VMEM((RING, BPW, EMB), jnp.float32),
        pltpu.VMEM((BPW, EMB), jnp.float32),
    ] + [pltpu.SemaphoreType.DMA] * RING,
    compiler_params=pltpu.CompilerParams(use_tc_tiling_on_sc=False),
)
def _mean_pool(x_hbm, tab_hbm, m_hbm, idx_v, rows_v, acc_v, *sems):
    _mean_pool_body(x_hbm, tab_hbm, m_hbm, idx_v, rows_v, acc_v, *sems)


def _mlp_body(m_ref, w1_ref, b1_ref, w2_ref, b2_ref, o_ref):
    h = jnp.dot(m_ref[...], w1_ref[...], preferred_element_type=jnp.float32)
    h = jnp.maximum(h + b1_ref[...], 0.0)
    o_ref[...] = jnp.dot(h, w2_ref[...],
                         preferred_element_type=jnp.float32) + b2_ref[...]


_mlp = pl.pallas_call(
    _mlp_body,
    out_shape=jax.ShapeDtypeStruct((B, OUT), jnp.float32),
)


def kernel(x, emb, W1, b1, W2, b2):
    packed = _tpack(emb.T)
    tab = packed.reshape(PACK_V, EMB)
    m = _mean_pool(x, tab)
    return _mlp(m, W1, b1.reshape(1, HID), W2, b2.reshape(1, OUT))


# speedup vs baseline: 2.3900x; 1.0192x over previous
import functools

import jax
import jax.numpy as jnp
from jax import lax
from jax.experimental import pallas as pl
from jax.experimental.pallas import tpu as pltpu
from jax.experimental.pallas import tpu_sc as plsc

VOCAB = 1000000
EMB = 64
HID = 128
OUT = 2
L, B = 200, 4096

NC, NS = 2, 16
NW = NC * NS
BPW = B // NW
LANES = 16
INV_L = 1.0 / L

CBLK = 32768
PU = CBLK
HU = PU // 2
TGRID = (VOCAB + CBLK - 1) // CBLK
PACK_ROWS = TGRID * (CBLK // 2)
PACK_V = PACK_ROWS * 2


def _tpack_body(a_ref, o_ref):
    a = a_ref[...]
    c = jnp.concatenate([a[:, : CBLK // 2], a[:, CBLK // 2 :]], axis=0)
    o_ref[...] = c.T


_tpack = pl.pallas_call(
    _tpack_body,
    grid=(TGRID,),
    in_specs=[pl.BlockSpec((EMB, CBLK), lambda i: (0, i))],
    out_specs=pl.BlockSpec((CBLK // 2, 2 * EMB), lambda i: (i, 0)),
    out_shape=jax.ShapeDtypeStruct((PACK_ROWS, 2 * EMB), jnp.float32),
)


RING = 4


def _mean_pool_body(x_hbm, tab_hbm, m_hbm, idx_v, rows_v, acc_v, *sems):
    wid = lax.axis_index("s") * NC + lax.axis_index("c")
    base = wid * BPW

    pltpu.sync_copy(x_hbm.at[:, pl.ds(base, BPW)], idx_v)

    def remap(l, carry):
        for c in range(BPW // LANES):
            sl = pl.ds(c * LANES, LANES)
            v = idx_v[l, sl]
            q = v & (PU - 1)
            idx_v[l, sl] = v + q + jnp.where(q >= HU, 1 - PU, 0)
        return carry

    lax.fori_loop(0, L, remap, 0)

    def issue(l, s):
        pltpu.async_copy(tab_hbm.at[idx_v.at[l]], rows_v.at[s], sems[s])

    def drain(s):
        pltpu.make_async_copy(tab_hbm.at[pl.ds(0, BPW)],
                              rows_v.at[s], sems[s]).wait()

    def zero(r, carry):
        for c in range(EMB // LANES):
            acc_v[r, pl.ds(c * LANES, LANES)] = jnp.zeros((LANES,),
                                                          jnp.float32)
        return carry

    lax.fori_loop(0, BPW, zero, 0)

    for s in range(RING):
        issue(s, s)

    def outer(i, carry):
        for s in range(RING):
            l = i * RING + s
            drain(s)

            def red(r, carry2):
                for c in range(EMB // LANES):
                    plsc.addupdate(acc_v.at[r, pl.ds(c * LANES, LANES)],
                                   rows_v[s, r, pl.ds(c * LANES, LANES)])
                return carry2

            lax.fori_loop(0, BPW, red, 0)

            nl = l + RING

            @pl.when(nl < L)
            def _():
                issue(nl, s)
        return carry

    lax.fori_loop(0, L // RING, outer, 0)

    def scale(r, carry):
        for c in range(EMB // LANES):
            sl = pl.ds(c * LANES, LANES)
            acc_v[r, sl] = acc_v[r, sl] * INV_L
        return carry

    lax.fori_loop(0, BPW, scale, 0)
    pltpu.sync_copy(acc_v, m_hbm.at[pl.ds(base, BPW)])


@functools.partial(
    pl.kernel,
    out_type=jax.ShapeDtypeStruct((B, EMB), jnp.float32),
    mesh=plsc.VectorSubcoreMesh(core_axis_name="c", subcore_axis_name="s"),
    scratch_types=[
        pltpu.VMEM((L, BPW), jnp.int32),
        pltpu.VMEM((RING, BPW, EMB), jnp.float32),
        pltpu.VMEM((BPW, EMB), jnp.float32),
    ] + [pltpu.SemaphoreType.DMA] * RING,
    compiler_params=pltpu.CompilerParams(use_tc_tiling_on_sc=False),
)
def _mean_pool(x_hbm, tab_hbm, m_hbm, idx_v, rows_v, acc_v, *sems):
    _mean_pool_body(x_hbm, tab_hbm, m_hbm, idx_v, rows_v, acc_v, *sems)


def _mlp_body(m_ref, w1_ref, b1_ref, w2_ref, b2_ref, o_ref):
    h = jnp.dot(m_ref[...], w1_ref[...], preferred_element_type=jnp.float32)
    h = jnp.maximum(h + b1_ref[...], 0.0)
    o_ref[...] = jnp.dot(h, w2_ref[...],
                         preferred_element_type=jnp.float32) + b2_ref[...]


_mlp = pl.pallas_call(
    _mlp_body,
    out_shape=jax.ShapeDtypeStruct((B, OUT), jnp.float32),
)


def kernel(x, emb, W1, b1, W2, b2):
    packed = _tpack(emb.T)
    tab = packed.reshape(PACK_V, EMB)
    m = _mean_pool(x, tab)
    return _mlp(m, W1, b1.reshape(1, HID), W2, b2.reshape(1, OUT))
